# baseline (device time: 423318 ns/iter reference)
import jax
import jax.numpy as jnp
from jax import lax
from jax.experimental import pallas as pl
from jax.experimental.pallas import tpu as pltpu

N_DEV = 32


def kernel(x, w_mat):
    m_total, k_per = x.shape
    _, n_cols = w_mat.shape
    m_per = m_total // N_DEV

    def body(x_ref, w_ref, out_ref, send_buf, recv_buf,
             send_sems, recv_sems, credit_sem):
        my = lax.axis_index("i")
        left = lax.rem(my + N_DEV - 1, N_DEV)
        right = lax.rem(my + 1, N_DEV)

        barrier = pltpu.get_barrier_semaphore()
        for nbr in (left, right):
            pl.semaphore_signal(barrier, inc=1, device_id=(nbr,),
                                device_id_type=pl.DeviceIdType.MESH)
        pl.semaphore_wait(barrier, 2)

        w_bf = w_ref[...].astype(jnp.bfloat16)

        def partial_chunk(c):
            xa = x_ref[pl.ds(c * m_per, m_per), :].astype(jnp.bfloat16)
            return jnp.dot(xa, w_bf, preferred_element_type=jnp.float32)

        for s in range(N_DEV - 1):
            slot = s % 2
            c = lax.rem(my - 1 - s + 2 * N_DEV, N_DEV)
            p = partial_chunk(c)
            if s == 0:
                send_buf[slot, :, :] = p
            else:
                send_buf[slot, :, :] = recv_buf[(s - 1) % 2, :, :] + p
                pl.semaphore_signal(credit_sem, inc=1, device_id=(left,),
                                    device_id_type=pl.DeviceIdType.MESH)
            if s >= 2:
                pl.semaphore_wait(credit_sem, 1)
            rdma = pltpu.make_async_remote_copy(
                src_ref=send_buf.at[slot],
                dst_ref=recv_buf.at[slot],
                send_sem=send_sems.at[slot],
                recv_sem=recv_sems.at[slot],
                device_id=(right,),
                device_id_type=pl.DeviceIdType.MESH,
            )
            rdma.start()
            rdma.wait()

        out_ref[...] = recv_buf[(N_DEV - 2) % 2, :, :] + partial_chunk(my)
        pl.semaphore_wait(credit_sem, 1)

    return pl.pallas_call(
        body,
        out_shape=jax.ShapeDtypeStruct((m_per, n_cols), jnp.float32),
        in_specs=[pl.BlockSpec(memory_space=pltpu.VMEM)] * 2,
        out_specs=pl.BlockSpec(memory_space=pltpu.VMEM),
        scratch_shapes=[
            pltpu.VMEM((2, m_per, n_cols), jnp.float32),
            pltpu.VMEM((2, m_per, n_cols), jnp.float32),
            pltpu.SemaphoreType.DMA((2,)),
            pltpu.SemaphoreType.DMA((2,)),
            pltpu.SemaphoreType.REGULAR,
        ],
        compiler_params=pltpu.CompilerParams(collective_id=0),
    )(x, w_mat)


# device time: 195399 ns/iter; 2.1664x vs baseline; 2.1664x over previous
import jax
import jax.numpy as jnp
from jax import lax
from jax.experimental import pallas as pl
from jax.experimental.pallas import tpu as pltpu

N_DEV = 32
K = 4
S = 4
NSTEP = N_DEV - 1


def kernel(x, w_mat):
    m_total, k_per = x.shape
    _, n_cols = w_mat.shape
    m_per = m_total // N_DEV
    half = n_cols // 2
    seg = half // K

    def body(x_ref, w_ref, out_ref,
             sb_r, rb_r, ss_r, rs_r, cr_r,
             sb_l, rb_l, ss_l, rs_l, cr_l):
        my = lax.axis_index("i")
        left = lax.rem(my + N_DEV - 1, N_DEV)
        right = lax.rem(my + 1, N_DEV)

        barrier = pltpu.get_barrier_semaphore()
        for nbr in (left, right):
            pl.semaphore_signal(barrier, inc=1, device_id=(nbr,),
                                device_id_type=pl.DeviceIdType.MESH)
        pl.semaphore_wait(barrier, 2)

        w_bf = w_ref[...].astype(jnp.bfloat16)

        def partial_half(c, lo):
            xa = x_ref[pl.ds(c * m_per, m_per), :].astype(jnp.bfloat16)
            return jnp.dot(xa, w_bf[:, lo:lo + half],
                           preferred_element_type=jnp.float32)

        dirs = (
            (sb_r, rb_r, ss_r, rs_r, cr_r, right, left, 0),
            (sb_l, rb_l, ss_l, rs_l, cr_l, left, right, half),
        )

        def mk(d, j, s):
            sb, rb, ss, rs, _, dst, _, _ = d
            return pltpu.make_async_remote_copy(
                src_ref=sb.at[j, s % 2],
                dst_ref=rb.at[j, s % S],
                send_sem=ss.at[j, s % 2],
                recv_sem=rs.at[j, s % S],
                device_id=(dst,),
                device_id_type=pl.DeviceIdType.MESH,
            )

        for s in range(NSTEP):
            for di, d in enumerate(dirs):
                sb, rb, ss, rs, cr, dst, ups, lo = d
                c = lax.rem(my - 1 - s + 2 * N_DEV, N_DEV) if di == 0 \
                    else lax.rem(my + 1 + s, N_DEV)
                p = partial_half(c, lo)
                for j in range(K):
                    cs = slice(j * seg, (j + 1) * seg)
                    if s == 0:
                        msg = p[:, cs]
                    else:
                        mk(d, j, s - 1).wait_recv()
                        msg = rb[j, (s - 1) % S].astype(jnp.float32) + p[:, cs]
                        pl.semaphore_signal(
                            cr.at[j], inc=1, device_id=(ups,),
                            device_id_type=pl.DeviceIdType.MESH)
                    if s >= S:
                        pl.semaphore_wait(cr.at[j], 1)
                    if s >= 2:
                        mk(d, j, s - 2).wait_send()
                    sb[j, s % 2] = msg.astype(jnp.bfloat16)
                    mk(d, j, s).start()

        for d in dirs:
            sb, rb, ss, rs, cr, dst, ups, lo = d
            p = partial_half(my, lo)
            for j in range(K):
                cs = slice(j * seg, (j + 1) * seg)
                mk(d, j, NSTEP - 1).wait_recv()
                out_ref[:, lo + j * seg:lo + (j + 1) * seg] = (
                    rb[j, (NSTEP - 1) % S].astype(jnp.float32) + p[:, cs])
                mk(d, j, NSTEP - 2).wait_send()
                mk(d, j, NSTEP - 1).wait_send()
                pl.semaphore_wait(cr.at[j], S - 1)

    return pl.pallas_call(
        body,
        out_shape=jax.ShapeDtypeStruct((m_per, n_cols), jnp.float32),
        in_specs=[pl.BlockSpec(memory_space=pltpu.VMEM)] * 2,
        out_specs=pl.BlockSpec(memory_space=pltpu.VMEM),
        scratch_shapes=[
            pltpu.VMEM((K, 2, m_per, seg), jnp.bfloat16),
            pltpu.VMEM((K, S, m_per, seg), jnp.bfloat16),
            pltpu.SemaphoreType.DMA((K, 2)),
            pltpu.SemaphoreType.DMA((K, S)),
            pltpu.SemaphoreType.REGULAR((K,)),
            pltpu.VMEM((K, 2, m_per, seg), jnp.bfloat16),
            pltpu.VMEM((K, S, m_per, seg), jnp.bfloat16),
            pltpu.SemaphoreType.DMA((K, 2)),
            pltpu.SemaphoreType.DMA((K, S)),
            pltpu.SemaphoreType.REGULAR((K,)),
        ],
        compiler_params=pltpu.CompilerParams(collective_id=0),
    )(x, w_mat)
